# baseline (speedup 1.0000x reference)
import functools

import jax
import jax.numpy as jnp
from jax import lax
from jax.experimental import pallas as pl
from jax.experimental.pallas import tpu as pltpu
from jax.experimental.pallas import tpu_sc as plsc

_NC = 2
_NS = 16
_K = 80
_SLOPE = 0.2


def _lrelu(t):
    return jnp.where(t >= 0, t, _SLOPE * t)


def _a1_body(x_ref, w_ref, ms_ref, md_ref, *out_refs):
    x = x_ref[...]
    xp = jnp.dot(x, w_ref[...], preferred_element_type=jnp.float32)
    for j in range(8):
        out_refs[j][...] = xp[:, j * 64:(j + 1) * 64]
    a_s = jnp.dot(xp, ms_ref[...], preferred_element_type=jnp.float32)
    a_d = jnp.dot(xp, md_ref[...], preferred_element_type=jnp.float32)
    pad = jnp.zeros((x.shape[0], 16 - a_s.shape[1]), jnp.float32)
    out_refs[8][...] = jnp.concatenate([a_s, pad], axis=1)
    out_refs[9][...] = jnp.concatenate([a_d, pad], axis=1)


def _a2_body(ea_ref, we_ref, me_ref, aep_ref):
    ve = jnp.dot(we_ref[...], me_ref[...], preferred_element_type=jnp.float32)
    ae = jnp.dot(ea_ref[...], ve, preferred_element_type=jnp.float32)
    pad = jnp.zeros((ae.shape[0], 16 - 2 * ae.shape[1]), jnp.float32)
    aep_ref[...] = jnp.concatenate([ae, ae, pad], axis=1)


def _b_body(h, p0_ref, p1_ref, asp_ref, adp_ref, *refs):
    xps = refs[:8]
    dinvp_ref, oloop_ref = refs[8], refs[9]
    acc = p0_ref[...] + p1_ref[...]
    ssum = acc[:, 0:h]
    aesum = acc[:, h:2 * h]
    cnt = acc[:, 2 * h:2 * h + 1]
    ae_loop = aesum / jnp.maximum(cnt, 1.0)
    tl = asp_ref[:, 0:h] + adp_ref[:, 0:h] + ae_loop
    s_loop = jnp.exp(_lrelu(tl))
    denom = ssum + s_loop
    dinv = 1.0 / (denom + 1e-16)
    wl = s_loop * dinv
    pad = jnp.zeros((acc.shape[0], 16 - h), jnp.float32)
    dinvp_ref[...] = jnp.concatenate([dinv, pad], axis=1)
    o_lo = wl[:, 0:1] * xps[0][...]
    o_hi = wl[:, 0:1] * xps[1][...]
    for hh in range(1, h):
        o_lo = o_lo + wl[:, hh:hh + 1] * xps[2 * hh][...]
        o_hi = o_hi + wl[:, hh:hh + 1] * xps[2 * hh + 1][...]
    oloop_ref[...] = jnp.concatenate([o_lo, o_hi], axis=1)


def _c_body(inv_h, x_ref, pl0_ref, pl1_ref, ph0_ref, ph1_ref, oloop_ref,
            bias_ref, out_ref):
    lo = pl0_ref[...] + pl1_ref[...]
    hi = ph0_ref[...] + ph1_ref[...]
    p = jnp.concatenate([lo, hi], axis=1) + oloop_ref[...]
    out_ref[...] = x_ref[...] + p * inv_h + bias_ref[...]


def _sc1_body(n, e, src_hbm, dst_hbm, aep_hbm, asp_hbm, adp_hbm,
              sflat_hbm, part1_hbm,
              bufs, stage_v, acc_sh, sems):
    cid = lax.axis_index("c")
    sid = lax.axis_index("s")
    tid = sid * _NC + cid
    ept = e // (_NC * _NS)
    nchunk = ept // _K
    nzw = 10
    rows_w = n // nzw
    ln = lax.iota(jnp.int32, 16)
    sem_g, sem_s = sems

    @pl.when(sid < nzw)
    def _zero():
        def _zrow(r, _):
            stage_v[r, :] = jnp.zeros((16,), jnp.float32)
            return 0
        lax.fori_loop(0, rows_w, _zrow, 0)
        pltpu.sync_copy(stage_v, acc_sh.at[pl.ds(sid * rows_w, rows_w)])
    plsc.subcore_barrier()

    def _lin(j, b):
        base = tid * ept + j * _K
        src_v, dst_v, aep_v, _, _, _, _, _ = bufs[b]
        pltpu.sync_copy(src_hbm.at[pl.ds(base, _K)], src_v)
        pltpu.sync_copy(dst_hbm.at[pl.ds(base, _K)], dst_v)
        pltpu.sync_copy(aep_hbm.at[pl.ds(base, _K)], aep_v)

    def _gath_start(b):
        src_v, dst_v, _, as_v, ad_v, _, _, _ = bufs[b]
        pltpu.async_copy(asp_hbm.at[src_v], as_v, sem_g[b])
        pltpu.async_copy(adp_hbm.at[dst_v], ad_v, sem_g[b])

    def _gath_wait(b):
        src_v, dst_v, _, as_v, ad_v, _, _, _ = bufs[b]
        pltpu.make_async_copy(asp_hbm.at[src_v], as_v, sem_g[b]).wait()
        pltpu.make_async_copy(adp_hbm.at[dst_v], ad_v, sem_g[b]).wait()

    def _compute(j, b):
        base = tid * ept + j * _K
        src_v, dst_v, aep_v, as_v, ad_v, scat_v, sflat_v, dsc_v = bufs[b]

        def _edge(k, _):
            aep = aep_v[k, :]
            t = as_v[k, :] + ad_v[k, :] + aep
            s16 = jnp.exp(_lrelu(t))
            ones16 = jnp.full((16,), 1.0, dtype=jnp.float32)
            zero16 = jnp.zeros((16,), jnp.float32)
            row = jnp.where(ln < 4, s16,
                            jnp.where(ln < 8, aep,
                                      jnp.where(ln == 8, ones16, zero16)))
            scat_v[k, :] = row
            return 0
        lax.fori_loop(0, _K, _edge, 0)

        def _pack(g, _):
            ridx = g * 4 + (ln >> 2)
            cidx = ln & 3
            sflat_v[pl.ds(g * 16, 16)] = plsc.load_gather(scat_v, [ridx, cidx])
            return 0
        lax.fori_loop(0, _K // 4, _pack, 0)

        pltpu.sync_copy(sflat_v, sflat_hbm.at[pl.ds(base * 4, _K * 4)])
        pltpu.sync_copy(scat_v, acc_sh.at[dst_v], add=True)

    _lin(0, 0)
    _gath_start(0)

    def _iter(i, _):
        for b in range(2):
            j = 2 * i + b

            @pl.when(j + 1 < nchunk)
            def _():
                _lin(j + 1, 1 - b)
                _gath_start(1 - b)

            @pl.when(j < nchunk)
            def _():
                _gath_wait(b)
                _compute(j, b)
        return 0
    lax.fori_loop(0, (nchunk + 1) // 2, _iter, 0)

    plsc.subcore_barrier()

    @pl.when(sid < nzw)
    def _read():
        r0 = sid * rows_w
        pltpu.sync_copy(acc_sh.at[pl.ds(r0, rows_w)], stage_v)
        pltpu.sync_copy(stage_v, part1_hbm.at[pl.ds(cid * n + r0, rows_w)])


def _sc2_body(n, e, src_hbm, dst_hbm, sflat_hbm, dinvp_hbm, xpcat_hbm,
              pout_hbm,
              bufs, zstage_v, out_sh, sems):
    cid = lax.axis_index("c")
    sid = lax.axis_index("s")
    tid = sid * _NC + cid
    ept = e // (_NC * _NS)
    nchunk = ept // _K
    nzw = 10
    rows_w = n // nzw
    zrows = zstage_v.shape[0]
    ln = lax.iota(jnp.int32, 16)
    sem_g, sem_s = sems

    def _lin(j, b):
        base = tid * ept + j * _K
        src_v, dst_v, _, _, sflat_v, _, _, _, _ = bufs[b]
        pltpu.sync_copy(src_hbm.at[pl.ds(base, _K)], src_v)
        pltpu.sync_copy(dst_hbm.at[pl.ds(base, _K)], dst_v)
        pltpu.sync_copy(sflat_hbm.at[pl.ds(base * 4, _K * 4)], sflat_v)

    def _gath_start(p, b):
        src_v, dst_v, idx4_v, _, _, _, dinv_v, rows, _ = bufs[b]
        pltpu.async_copy(dinvp_hbm.at[dst_v], dinv_v, sem_g[b])

        def _ix(g, _):
            v = src_v[pl.ds(g * 16, 16)]
            for h in range(4):
                idx4_v[pl.ds(h * _K + g * 16, 16)] = v + (2 * h + p) * n
            return 0
        lax.fori_loop(0, _K // 16, _ix, 0)
        for h in range(4):
            pltpu.async_copy(xpcat_hbm.at[idx4_v.at[pl.ds(h * _K, _K)]],
                             rows[h], sem_g[b])

    def _gath_wait(p, b):
        _, dst_v, idx4_v, _, _, _, dinv_v, rows, _ = bufs[b]
        pltpu.make_async_copy(dinvp_hbm.at[dst_v], dinv_v, sem_g[b]).wait()
        for h in range(4):
            pltpu.make_async_copy(xpcat_hbm.at[idx4_v.at[pl.ds(h * _K, _K)]],
                                  rows[h], sem_g[b]).wait()

    def _scat_wait(b):
        _, _, _, dsc_v, _, _, _, _, msg_v = bufs[b]
        pltpu.make_async_copy(msg_v, out_sh.at[dsc_v], sem_s[b]).wait()

    def _compute(b, not_first):
        src_v, dst_v, _, dsc_v, sflat_v, w_v, dinv_v, rows, msg_v = bufs[b]

        def _wg(g, _):
            ridx = g * 4 + (ln >> 2)
            hidx = ln & 3
            dv = plsc.load_gather(dinv_v, [ridx, hidx])
            w_v[pl.ds(g * 16, 16)] = sflat_v[pl.ds(g * 16, 16)] * dv
            return 0
        lax.fori_loop(0, _K // 4, _wg, 0)

        @pl.when(not_first)
        def _():
            _scat_wait(b)

        def _dcp(g, _):
            dsc_v[pl.ds(g * 16, 16)] = dst_v[pl.ds(g * 16, 16)]
            return 0
        lax.fori_loop(0, _K // 16, _dcp, 0)

        def _edge(k, _):
            acc = [jnp.zeros((16,), jnp.float32) for _ in range(4)]
            for h in range(4):
                wvec = plsc.load_gather(
                    w_v, [jnp.full((16,), 4 * k + h, dtype=jnp.int32)])
                for g in range(4):
                    acc[g] = acc[g] + wvec * rows[h][k, pl.ds(g * 16, 16)]
            for g in range(4):
                msg_v[k, pl.ds(g * 16, 16)] = acc[g]
            return 0
        lax.fori_loop(0, _K, _edge, 0)
        pltpu.async_copy(msg_v, out_sh.at[dsc_v], sem_s[b], add=True)

    def _phase(p, _):
        @pl.when(sid < nzw)
        def _zero():
            def _zrow(r, _):
                for g in range(4):
                    zstage_v[r, pl.ds(g * 16, 16)] = jnp.zeros((16,),
                                                               jnp.float32)
                return 0
            lax.fori_loop(0, zrows, _zrow, 0)
            for j in range(rows_w // zrows):
                pltpu.sync_copy(
                    zstage_v,
                    out_sh.at[pl.ds(sid * rows_w + j * zrows, zrows)])
        plsc.subcore_barrier()

        _lin(0, 0)
        _gath_start(p, 0)

        def _iter(i, _):
            for b in range(2):
                j = 2 * i + b

                @pl.when(j + 1 < nchunk)
                def _():
                    _lin(j + 1, 1 - b)
                    _gath_start(p, 1 - b)

                @pl.when(j < nchunk)
                def _():
                    _gath_wait(p, b)
                    _compute(b, j >= 2)
            return 0
        lax.fori_loop(0, (nchunk + 1) // 2, _iter, 0)
        _scat_wait(0)
        _scat_wait(1)

        plsc.subcore_barrier()

        @pl.when(sid < nzw)
        def _read():
            rr = sid * rows_w
            for j in range(rows_w // zrows):
                pltpu.sync_copy(out_sh.at[pl.ds(rr + j * zrows, zrows)],
                                zstage_v)
                ro = (p * _NC + cid) * n + rr + j * zrows
                pltpu.sync_copy(zstage_v, pout_hbm.at[pl.ds(ro, zrows)])
        plsc.subcore_barrier()
        return 0
    lax.fori_loop(0, 2, _phase, 0)


def kernel(x, edge_index, edge_attr, W, att_src, att_dst, W_edge, att_edge,
           bias):
    n, d = x.shape
    e = edge_index.shape[1]
    h = att_src.shape[1]
    c = att_src.shape[2]
    ed = W_edge.shape[0]
    assert h == 4 and c == 128 and d == 128
    assert e % (_NC * _NS * _K) == 0 and n % 2000 == 0

    src = edge_index[0]
    dst = edge_index[1]

    eye_h = jnp.eye(h, dtype=jnp.float32)
    m_src = (att_src[0][:, :, None] * eye_h[:, None, :]).reshape(h * c, h)
    m_dst = (att_dst[0][:, :, None] * eye_h[:, None, :]).reshape(h * c, h)
    m_edge = (att_edge[0][:, :, None] * eye_h[:, None, :]).reshape(h * c, h)

    bn = 1000
    gn = n // bn
    be = e // 40
    ge = e // be

    a1_out = pl.pallas_call(
        _a1_body,
        grid=(gn,),
        in_specs=[
            pl.BlockSpec((bn, d), lambda i: (i, 0)),
            pl.BlockSpec((d, h * c), lambda i: (0, 0)),
            pl.BlockSpec((h * c, h), lambda i: (0, 0)),
            pl.BlockSpec((h * c, h), lambda i: (0, 0)),
        ],
        out_specs=[pl.BlockSpec((bn, 64), lambda i: (i, 0))] * 8
        + [pl.BlockSpec((bn, 16), lambda i: (i, 0))] * 2,
        out_shape=[jax.ShapeDtypeStruct((n, 64), jnp.float32)] * 8
        + [jax.ShapeDtypeStruct((n, 16), jnp.float32)] * 2,
    )(x, W, m_src, m_dst)
    xps = a1_out[:8]
    asp, adp = a1_out[8], a1_out[9]

    aep = pl.pallas_call(
        _a2_body,
        grid=(ge,),
        in_specs=[
            pl.BlockSpec((be, ed), lambda i: (i, 0)),
            pl.BlockSpec((ed, h * c), lambda i: (0, 0)),
            pl.BlockSpec((h * c, h), lambda i: (0, 0)),
        ],
        out_specs=pl.BlockSpec((be, 16), lambda i: (i, 0)),
        out_shape=jax.ShapeDtypeStruct((e, 16), jnp.float32),
    )(edge_attr, W_edge, m_edge)

    mesh = plsc.VectorSubcoreMesh(core_axis_name="c", subcore_axis_name="s")
    sc_params = pltpu.CompilerParams(use_tc_tiling_on_sc=False,
                                     needs_layout_passes=False)

    sflat, part1 = pl.kernel(
        functools.partial(_sc1_body, n, e),
        out_type=(
            jax.ShapeDtypeStruct((4 * e,), jnp.float32),
            jax.ShapeDtypeStruct((_NC * n, 16), jnp.float32),
        ),
        mesh=mesh,
        compiler_params=sc_params,
        scratch_types=[
            tuple(
                (pltpu.VMEM((_K,), jnp.int32),
                 pltpu.VMEM((_K,), jnp.int32),
                 pltpu.VMEM((_K, 16), jnp.float32),
                 pltpu.VMEM((_K, 16), jnp.float32),
                 pltpu.VMEM((_K, 16), jnp.float32),
                 pltpu.VMEM((_K, 16), jnp.float32),
                 pltpu.VMEM((4 * _K,), jnp.float32),
                 pltpu.VMEM((_K,), jnp.int32))
                for _ in range(2)),
            pltpu.VMEM((n // 10, 16), jnp.float32),
            pltpu.MemorySpace.VMEM_SHARED((n, 16), jnp.float32),
            ((pltpu.SemaphoreType.DMA, pltpu.SemaphoreType.DMA),
             (pltpu.SemaphoreType.DMA, pltpu.SemaphoreType.DMA)),
        ],
    )(src, dst, aep, asp, adp)

    dinvp, oloop = pl.pallas_call(
        functools.partial(_b_body, h),
        grid=(gn,),
        in_specs=[
            pl.BlockSpec((bn, 16), lambda i: (i, 0)),
            pl.BlockSpec((bn, 16), lambda i: (i + gn, 0)),
            pl.BlockSpec((bn, 16), lambda i: (i, 0)),
            pl.BlockSpec((bn, 16), lambda i: (i, 0)),
        ] + [pl.BlockSpec((bn, 64), lambda i: (i, 0))] * 8,
        out_specs=[
            pl.BlockSpec((bn, 16), lambda i: (i, 0)),
            pl.BlockSpec((bn, c), lambda i: (i, 0)),
        ],
        out_shape=[
            jax.ShapeDtypeStruct((n, 16), jnp.float32),
            jax.ShapeDtypeStruct((n, c), jnp.float32),
        ],
    )(part1, part1, asp, adp, *xps)

    xpcat = jnp.concatenate(xps, axis=0)
    pout = pl.kernel(
        functools.partial(_sc2_body, n, e),
        out_type=jax.ShapeDtypeStruct((4 * n, 64), jnp.float32),
        mesh=mesh,
        compiler_params=sc_params,
        scratch_types=[
            tuple(
                (pltpu.VMEM((_K,), jnp.int32),
                 pltpu.VMEM((_K,), jnp.int32),
                 pltpu.VMEM((4 * _K,), jnp.int32),
                 pltpu.VMEM((_K,), jnp.int32),
                 pltpu.VMEM((4 * _K,), jnp.float32),
                 pltpu.VMEM((4 * _K,), jnp.float32),
                 pltpu.VMEM((_K, 16), jnp.float32),
                 tuple(pltpu.VMEM((_K, 64), jnp.float32)
                       for _ in range(4)),
                 pltpu.VMEM((_K, 64), jnp.float32))
                for _ in range(2)),
            pltpu.VMEM((200, 64), jnp.float32),
            pltpu.MemorySpace.VMEM_SHARED((n, 64), jnp.float32),
            ((pltpu.SemaphoreType.DMA, pltpu.SemaphoreType.DMA),
             (pltpu.SemaphoreType.DMA, pltpu.SemaphoreType.DMA)),
        ],
    )(src, dst, sflat, dinvp, xpcat)

    out = pl.pallas_call(
        functools.partial(_c_body, 1.0 / h),
        grid=(gn,),
        in_specs=[
            pl.BlockSpec((bn, d), lambda i: (i, 0)),
            pl.BlockSpec((bn, 64), lambda i: (i, 0)),
            pl.BlockSpec((bn, 64), lambda i: (i + gn, 0)),
            pl.BlockSpec((bn, 64), lambda i: (i + 2 * gn, 0)),
            pl.BlockSpec((bn, 64), lambda i: (i + 3 * gn, 0)),
            pl.BlockSpec((bn, c), lambda i: (i, 0)),
            pl.BlockSpec((1, d), lambda i: (0, 0)),
        ],
        out_specs=pl.BlockSpec((bn, d), lambda i: (i, 0)),
        out_shape=jax.ShapeDtypeStruct((n, d), jnp.float32),
    )(x, pout, pout, pout, pout, oloop, bias.reshape(1, d))

    return out


# speedup vs baseline: 1.1334x; 1.1334x over previous
import functools

import jax
import jax.numpy as jnp
from jax import lax
from jax.experimental import pallas as pl
from jax.experimental.pallas import tpu as pltpu
from jax.experimental.pallas import tpu_sc as plsc

_NC = 2
_NS = 16
_K = 80
_SLOPE = 0.2


def _lrelu(t):
    return jnp.where(t >= 0, t, _SLOPE * t)


def _a1_body(x_ref, w_ref, ms_ref, md_ref, *out_refs):
    x = x_ref[...]
    xp = jnp.dot(x, w_ref[...], preferred_element_type=jnp.float32)
    for j in range(8):
        out_refs[j][...] = xp[:, j * 64:(j + 1) * 64]
    a_s = jnp.dot(xp, ms_ref[...], preferred_element_type=jnp.float32)
    a_d = jnp.dot(xp, md_ref[...], preferred_element_type=jnp.float32)
    pad = jnp.zeros((x.shape[0], 16 - a_s.shape[1]), jnp.float32)
    out_refs[8][...] = jnp.concatenate([a_s, pad], axis=1)
    out_refs[9][...] = jnp.concatenate([a_d, pad], axis=1)


def _a2_body(ea_ref, we_ref, me_ref, aep_ref):
    ve = jnp.dot(we_ref[...], me_ref[...], preferred_element_type=jnp.float32)
    ae = jnp.dot(ea_ref[...], ve, preferred_element_type=jnp.float32)
    pad = jnp.zeros((ae.shape[0], 16 - 2 * ae.shape[1]), jnp.float32)
    aep_ref[...] = jnp.concatenate([ae, ae, pad], axis=1)


def _b_body(h, p0_ref, p1_ref, asp_ref, adp_ref, *refs):
    xps = refs[:8]
    dinvp_ref, oloop_ref = refs[8], refs[9]
    acc = p0_ref[...] + p1_ref[...]
    ssum = acc[:, 0:h]
    aesum = acc[:, h:2 * h]
    cnt = acc[:, 2 * h:2 * h + 1]
    ae_loop = aesum / jnp.maximum(cnt, 1.0)
    tl = asp_ref[:, 0:h] + adp_ref[:, 0:h] + ae_loop
    s_loop = jnp.exp(_lrelu(tl))
    denom = ssum + s_loop
    dinv = 1.0 / (denom + 1e-16)
    wl = s_loop * dinv
    pad = jnp.zeros((acc.shape[0], 16 - h), jnp.float32)
    dinvp_ref[...] = jnp.concatenate([dinv, pad], axis=1)
    o_lo = wl[:, 0:1] * xps[0][...]
    o_hi = wl[:, 0:1] * xps[1][...]
    for hh in range(1, h):
        o_lo = o_lo + wl[:, hh:hh + 1] * xps[2 * hh][...]
        o_hi = o_hi + wl[:, hh:hh + 1] * xps[2 * hh + 1][...]
    oloop_ref[...] = jnp.concatenate([o_lo, o_hi], axis=1)


def _c_body(inv_h, x_ref, pl0_ref, pl1_ref, ph0_ref, ph1_ref, oloop_ref,
            bias_ref, out_ref):
    lo = pl0_ref[...] + pl1_ref[...]
    hi = ph0_ref[...] + ph1_ref[...]
    p = jnp.concatenate([lo, hi], axis=1) + oloop_ref[...]
    out_ref[...] = x_ref[...] + p * inv_h + bias_ref[...]


def _sc1_body(n, e, ei_hbm, aep_hbm, asp_hbm, adp_hbm,
              sflat_hbm, part1_hbm,
              bufs, stage_v, acc_sh, sems):
    cid = lax.axis_index("c")
    sid = lax.axis_index("s")
    tid = sid * _NC + cid
    ept = e // (_NC * _NS)
    nchunk = ept // _K
    nzw = 10
    rows_w = n // nzw
    ln = lax.iota(jnp.int32, 16)
    sem_g, sem_s = sems

    @pl.when(sid < nzw)
    def _zero():
        def _zrow(r, _):
            stage_v[r, :] = jnp.zeros((16,), jnp.float32)
            return 0
        lax.fori_loop(0, rows_w, _zrow, 0)
        pltpu.sync_copy(stage_v, acc_sh.at[pl.ds(sid * rows_w, rows_w)])
    plsc.subcore_barrier()

    def _lin(j, b):
        base = tid * ept + j * _K
        ei_v, aep_v, _, _, _, _, _ = bufs[b]
        pltpu.sync_copy(ei_hbm.at[:, pl.ds(base, _K)], ei_v)
        pltpu.sync_copy(aep_hbm.at[pl.ds(base, _K)], aep_v)

    def _gath_start(b):
        ei_v, _, as_v, ad_v, _, _, _ = bufs[b]
        pltpu.async_copy(asp_hbm.at[ei_v.at[0]], as_v, sem_g[b])
        pltpu.async_copy(adp_hbm.at[ei_v.at[1]], ad_v, sem_g[b])

    def _gath_wait(b):
        ei_v, _, as_v, ad_v, _, _, _ = bufs[b]
        pltpu.make_async_copy(asp_hbm.at[ei_v.at[0]], as_v, sem_g[b]).wait()
        pltpu.make_async_copy(adp_hbm.at[ei_v.at[1]], ad_v, sem_g[b]).wait()

    def _compute(j, b):
        base = tid * ept + j * _K
        ei_v, aep_v, as_v, ad_v, scat_v, sflat_v, dsc_v = bufs[b]

        def _edge(k, _):
            aep = aep_v[k, :]
            t = as_v[k, :] + ad_v[k, :] + aep
            s16 = jnp.exp(_lrelu(t))
            ones16 = jnp.full((16,), 1.0, dtype=jnp.float32)
            zero16 = jnp.zeros((16,), jnp.float32)
            row = jnp.where(ln < 4, s16,
                            jnp.where(ln < 8, aep,
                                      jnp.where(ln == 8, ones16, zero16)))
            scat_v[k, :] = row
            return 0
        lax.fori_loop(0, _K, _edge, 0)

        def _pack(g, _):
            ridx = g * 4 + (ln >> 2)
            cidx = ln & 3
            sflat_v[pl.ds(g * 16, 16)] = plsc.load_gather(scat_v, [ridx, cidx])
            return 0
        lax.fori_loop(0, _K // 4, _pack, 0)

        pltpu.sync_copy(sflat_v, sflat_hbm.at[pl.ds(base * 4, _K * 4)])
        pltpu.sync_copy(scat_v, acc_sh.at[ei_v.at[1]], add=True)

    _lin(0, 0)
    _gath_start(0)

    def _iter(i, _):
        for b in range(2):
            j = 2 * i + b

            @pl.when(j + 1 < nchunk)
            def _():
                _lin(j + 1, 1 - b)
                _gath_start(1 - b)

            @pl.when(j < nchunk)
            def _():
                _gath_wait(b)
                _compute(j, b)
        return 0
    lax.fori_loop(0, (nchunk + 1) // 2, _iter, 0)

    plsc.subcore_barrier()

    @pl.when(sid < nzw)
    def _read():
        r0 = sid * rows_w
        pltpu.sync_copy(acc_sh.at[pl.ds(r0, rows_w)], stage_v)
        pltpu.sync_copy(stage_v, part1_hbm.at[pl.ds(cid * n + r0, rows_w)])


def _sc2_body(n, e, ei_hbm, sflat_hbm, dinvp_hbm, xpcat_hbm,
              pout_hbm,
              bufs, zstage_v, out_sh, sems):
    cid = lax.axis_index("c")
    sid = lax.axis_index("s")
    tid = sid * _NC + cid
    ept = e // (_NC * _NS)
    nchunk = ept // _K
    nzw = 10
    rows_w = n // nzw
    zrows = zstage_v.shape[0]
    ln = lax.iota(jnp.int32, 16)
    sem_g, sem_s = sems

    def _lin(j, b):
        base = tid * ept + j * _K
        ei_v, _, _, sflat_v, _, _, _, _ = bufs[b]
        pltpu.sync_copy(ei_hbm.at[:, pl.ds(base, _K)], ei_v)
        pltpu.sync_copy(sflat_hbm.at[pl.ds(base * 4, _K * 4)], sflat_v)

    def _gath_start(p, b):
        ei_v, idx4_v, _, _, _, dinv_v, rows, _ = bufs[b]
        pltpu.async_copy(dinvp_hbm.at[ei_v.at[1]], dinv_v, sem_g[b])

        def _ix(g, _):
            v = ei_v[0, pl.ds(g * 16, 16)]
            for h in range(4):
                idx4_v[pl.ds(h * _K + g * 16, 16)] = v + (2 * h + p) * n
            return 0
        lax.fori_loop(0, _K // 16, _ix, 0)
        for h in range(4):
            pltpu.async_copy(xpcat_hbm.at[idx4_v.at[pl.ds(h * _K, _K)]],
                             rows[h], sem_g[b])

    def _gath_wait(p, b):
        ei_v, idx4_v, _, _, _, dinv_v, rows, _ = bufs[b]
        pltpu.make_async_copy(dinvp_hbm.at[ei_v.at[1]], dinv_v,
                              sem_g[b]).wait()
        for h in range(4):
            pltpu.make_async_copy(xpcat_hbm.at[idx4_v.at[pl.ds(h * _K, _K)]],
                                  rows[h], sem_g[b]).wait()

    def _scat_wait(b):
        _, _, dsc_v, _, _, _, _, msg_v = bufs[b]
        pltpu.make_async_copy(msg_v, out_sh.at[dsc_v], sem_s[b]).wait()

    def _compute(b, not_first):
        ei_v, _, dsc_v, sflat_v, w_v, dinv_v, rows, msg_v = bufs[b]

        def _wg(g, _):
            ridx = g * 4 + (ln >> 2)
            hidx = ln & 3
            dv = plsc.load_gather(dinv_v, [ridx, hidx])
            w_v[pl.ds(g * 16, 16)] = sflat_v[pl.ds(g * 16, 16)] * dv
            return 0
        lax.fori_loop(0, _K // 4, _wg, 0)

        @pl.when(not_first)
        def _():
            _scat_wait(b)

        def _dcp(g, _):
            dsc_v[pl.ds(g * 16, 16)] = ei_v[1, pl.ds(g * 16, 16)]
            return 0
        lax.fori_loop(0, _K // 16, _dcp, 0)

        def _edge(k, _):
            acc = [jnp.zeros((16,), jnp.float32) for _ in range(4)]
            for h in range(4):
                wvec = plsc.load_gather(
                    w_v, [jnp.full((16,), 4 * k + h, dtype=jnp.int32)])
                for g in range(4):
                    acc[g] = acc[g] + wvec * rows[h][k, pl.ds(g * 16, 16)]
            for g in range(4):
                msg_v[k, pl.ds(g * 16, 16)] = acc[g]
            return 0
        lax.fori_loop(0, _K, _edge, 0)
        pltpu.async_copy(msg_v, out_sh.at[dsc_v], sem_s[b], add=True)

    def _phase(p, _):
        @pl.when(sid < nzw)
        def _zero():
            def _zrow(r, _):
                for g in range(4):
                    zstage_v[r, pl.ds(g * 16, 16)] = jnp.zeros((16,),
                                                               jnp.float32)
                return 0
            lax.fori_loop(0, zrows, _zrow, 0)
            for j in range(rows_w // zrows):
                pltpu.sync_copy(
                    zstage_v,
                    out_sh.at[pl.ds(sid * rows_w + j * zrows, zrows)])
        plsc.subcore_barrier()

        _lin(0, 0)
        _gath_start(p, 0)

        def _iter(i, _):
            for b in range(2):
                j = 2 * i + b

                @pl.when(j + 1 < nchunk)
                def _():
                    _lin(j + 1, 1 - b)
                    _gath_start(p, 1 - b)

                @pl.when(j < nchunk)
                def _():
                    _gath_wait(p, b)
                    _compute(b, j >= 2)
            return 0
        lax.fori_loop(0, (nchunk + 1) // 2, _iter, 0)
        _scat_wait(0)
        _scat_wait(1)

        plsc.subcore_barrier()

        @pl.when(sid < nzw)
        def _read():
            rr = sid * rows_w
            for j in range(rows_w // zrows):
                pltpu.sync_copy(out_sh.at[pl.ds(rr + j * zrows, zrows)],
                                zstage_v)
                ro = (p * _NC + cid) * n + rr + j * zrows
                pltpu.sync_copy(zstage_v, pout_hbm.at[pl.ds(ro, zrows)])
        plsc.subcore_barrier()
        return 0
    lax.fori_loop(0, 2, _phase, 0)


def kernel(x, edge_index, edge_attr, W, att_src, att_dst, W_edge, att_edge,
           bias):
    n, d = x.shape
    e = edge_index.shape[1]
    h = att_src.shape[1]
    c = att_src.shape[2]
    ed = W_edge.shape[0]
    assert h == 4 and c == 128 and d == 128
    assert e % (_NC * _NS * _K) == 0 and n % 2000 == 0

    eye_h = jnp.eye(h, dtype=jnp.float32)
    m_src = (att_src[0][:, :, None] * eye_h[:, None, :]).reshape(h * c, h)
    m_dst = (att_dst[0][:, :, None] * eye_h[:, None, :]).reshape(h * c, h)
    m_edge = (att_edge[0][:, :, None] * eye_h[:, None, :]).reshape(h * c, h)

    bn = 1000
    gn = n // bn
    be = e // 40
    ge = e // be

    a1_out = pl.pallas_call(
        _a1_body,
        grid=(gn,),
        in_specs=[
            pl.BlockSpec((bn, d), lambda i: (i, 0)),
            pl.BlockSpec((d, h * c), lambda i: (0, 0)),
            pl.BlockSpec((h * c, h), lambda i: (0, 0)),
            pl.BlockSpec((h * c, h), lambda i: (0, 0)),
        ],
        out_specs=[pl.BlockSpec((bn, 64), lambda i: (i, 0))] * 8
        + [pl.BlockSpec((bn, 16), lambda i: (i, 0))] * 2,
        out_shape=[jax.ShapeDtypeStruct((n, 64), jnp.float32)] * 8
        + [jax.ShapeDtypeStruct((n, 16), jnp.float32)] * 2,
    )(x, W, m_src, m_dst)
    xps = a1_out[:8]
    asp, adp = a1_out[8], a1_out[9]

    aep = pl.pallas_call(
        _a2_body,
        grid=(ge,),
        in_specs=[
            pl.BlockSpec((be, ed), lambda i: (i, 0)),
            pl.BlockSpec((ed, h * c), lambda i: (0, 0)),
            pl.BlockSpec((h * c, h), lambda i: (0, 0)),
        ],
        out_specs=pl.BlockSpec((be, 16), lambda i: (i, 0)),
        out_shape=jax.ShapeDtypeStruct((e, 16), jnp.float32),
    )(edge_attr, W_edge, m_edge)

    mesh = plsc.VectorSubcoreMesh(core_axis_name="c", subcore_axis_name="s")
    sc_params = pltpu.CompilerParams(use_tc_tiling_on_sc=False,
                                     needs_layout_passes=False)

    sflat, part1 = pl.kernel(
        functools.partial(_sc1_body, n, e),
        out_type=(
            jax.ShapeDtypeStruct((4 * e,), jnp.float32),
            jax.ShapeDtypeStruct((_NC * n, 16), jnp.float32),
        ),
        mesh=mesh,
        compiler_params=sc_params,
        scratch_types=[
            tuple(
                (pltpu.VMEM((2, _K), jnp.int32),
                 pltpu.VMEM((_K, 16), jnp.float32),
                 pltpu.VMEM((_K, 16), jnp.float32),
                 pltpu.VMEM((_K, 16), jnp.float32),
                 pltpu.VMEM((_K, 16), jnp.float32),
                 pltpu.VMEM((4 * _K,), jnp.float32),
                 pltpu.VMEM((_K,), jnp.int32))
                for _ in range(2)),
            pltpu.VMEM((n // 10, 16), jnp.float32),
            pltpu.MemorySpace.VMEM_SHARED((n, 16), jnp.float32),
            ((pltpu.SemaphoreType.DMA, pltpu.SemaphoreType.DMA),
             (pltpu.SemaphoreType.DMA, pltpu.SemaphoreType.DMA)),
        ],
    )(edge_index, aep, asp, adp)

    dinvp, oloop = pl.pallas_call(
        functools.partial(_b_body, h),
        grid=(gn,),
        in_specs=[
            pl.BlockSpec((bn, 16), lambda i: (i, 0)),
            pl.BlockSpec((bn, 16), lambda i: (i + gn, 0)),
            pl.BlockSpec((bn, 16), lambda i: (i, 0)),
            pl.BlockSpec((bn, 16), lambda i: (i, 0)),
        ] + [pl.BlockSpec((bn, 64), lambda i: (i, 0))] * 8,
        out_specs=[
            pl.BlockSpec((bn, 16), lambda i: (i, 0)),
            pl.BlockSpec((bn, c), lambda i: (i, 0)),
        ],
        out_shape=[
            jax.ShapeDtypeStruct((n, 16), jnp.float32),
            jax.ShapeDtypeStruct((n, c), jnp.float32),
        ],
    )(part1, part1, asp, adp, *xps)

    xpcat = jnp.concatenate(xps, axis=0)
    pout = pl.kernel(
        functools.partial(_sc2_body, n, e),
        out_type=jax.ShapeDtypeStruct((4 * n, 64), jnp.float32),
        mesh=mesh,
        compiler_params=sc_params,
        scratch_types=[
            tuple(
                (pltpu.VMEM((2, _K), jnp.int32),
                 pltpu.VMEM((4 * _K,), jnp.int32),
                 pltpu.VMEM((_K,), jnp.int32),
                 pltpu.VMEM((4 * _K,), jnp.float32),
                 pltpu.VMEM((4 * _K,), jnp.float32),
                 pltpu.VMEM((_K, 16), jnp.float32),
                 tuple(pltpu.VMEM((_K, 64), jnp.float32)
                       for _ in range(4)),
                 pltpu.VMEM((_K, 64), jnp.float32))
                for _ in range(2)),
            pltpu.VMEM((200, 64), jnp.float32),
            pltpu.MemorySpace.VMEM_SHARED((n, 64), jnp.float32),
            ((pltpu.SemaphoreType.DMA, pltpu.SemaphoreType.DMA),
             (pltpu.SemaphoreType.DMA, pltpu.SemaphoreType.DMA)),
        ],
    )(edge_index, sflat, dinvp, xpcat)

    out = pl.pallas_call(
        functools.partial(_c_body, 1.0 / h),
        grid=(gn,),
        in_specs=[
            pl.BlockSpec((bn, d), lambda i: (i, 0)),
            pl.BlockSpec((bn, 64), lambda i: (i, 0)),
            pl.BlockSpec((bn, 64), lambda i: (i + gn, 0)),
            pl.BlockSpec((bn, 64), lambda i: (i + 2 * gn, 0)),
            pl.BlockSpec((bn, 64), lambda i: (i + 3 * gn, 0)),
            pl.BlockSpec((bn, c), lambda i: (i, 0)),
            pl.BlockSpec((1, d), lambda i: (0, 0)),
        ],
        out_specs=pl.BlockSpec((bn, d), lambda i: (i, 0)),
        out_shape=jax.ShapeDtypeStruct((n, d), jnp.float32),
    )(x, pout, pout, pout, pout, oloop, bias.reshape(1, d))

    return out


# speedup vs baseline: 1.1527x; 1.0170x over previous
import functools

import jax
import jax.numpy as jnp
from jax import lax
from jax.experimental import pallas as pl
from jax.experimental.pallas import tpu as pltpu
from jax.experimental.pallas import tpu_sc as plsc

_NC = 2
_NS = 16
_K = 80
_SLOPE = 0.2


def _lrelu(t):
    return jnp.where(t >= 0, t, _SLOPE * t)


def _a1_body(x_ref, w_ref, ms_ref, md_ref, *out_refs):
    x = x_ref[...]
    xp = jnp.dot(x, w_ref[...], preferred_element_type=jnp.float32)
    for j in range(8):
        out_refs[j][...] = xp[:, j * 64:(j + 1) * 64]
    a_s = jnp.dot(xp, ms_ref[...], preferred_element_type=jnp.float32)
    a_d = jnp.dot(xp, md_ref[...], preferred_element_type=jnp.float32)
    pad = jnp.zeros((x.shape[0], 16 - a_s.shape[1]), jnp.float32)
    out_refs[8][...] = jnp.concatenate([a_s, pad], axis=1)
    out_refs[9][...] = jnp.concatenate([a_d, pad], axis=1)


def _a2_body(ea_ref, we_ref, me_ref, aep_ref):
    ve = jnp.dot(we_ref[...], me_ref[...], preferred_element_type=jnp.float32)
    ae = jnp.dot(ea_ref[...], ve, preferred_element_type=jnp.float32)
    pad = jnp.zeros((ae.shape[0], 16 - 2 * ae.shape[1]), jnp.float32)
    aep_ref[...] = jnp.concatenate([ae, ae, pad], axis=1)


def _b_body(h, p0_ref, p1_ref, asp_ref, adp_ref, *refs):
    xps = refs[:8]
    dinvp_ref, oloop_ref = refs[8], refs[9]
    acc = p0_ref[...] + p1_ref[...]
    ssum = acc[:, 0:h]
    aesum = acc[:, h:2 * h]
    cnt = acc[:, 2 * h:2 * h + 1]
    ae_loop = aesum / jnp.maximum(cnt, 1.0)
    tl = asp_ref[:, 0:h] + adp_ref[:, 0:h] + ae_loop
    s_loop = jnp.exp(_lrelu(tl))
    denom = ssum + s_loop
    dinv = 1.0 / (denom + 1e-16)
    wl = s_loop * dinv
    pad = jnp.zeros((acc.shape[0], 16 - h), jnp.float32)
    dinvp_ref[...] = jnp.concatenate([dinv, pad], axis=1)
    o_lo = wl[:, 0:1] * xps[0][...]
    o_hi = wl[:, 0:1] * xps[1][...]
    for hh in range(1, h):
        o_lo = o_lo + wl[:, hh:hh + 1] * xps[2 * hh][...]
        o_hi = o_hi + wl[:, hh:hh + 1] * xps[2 * hh + 1][...]
    oloop_ref[...] = jnp.concatenate([o_lo, o_hi], axis=1)


def _c_body(inv_h, x_ref, pl0_ref, pl1_ref, ph0_ref, ph1_ref, oloop_ref,
            bias_ref, out_ref):
    lo = pl0_ref[...] + pl1_ref[...]
    hi = ph0_ref[...] + ph1_ref[...]
    p = jnp.concatenate([lo, hi], axis=1) + oloop_ref[...]
    out_ref[...] = x_ref[...] + p * inv_h + bias_ref[...]


def _sc1_body(n, e, ei_hbm, aep_hbm, asp_hbm, adp_hbm,
              sflat_hbm, part1_hbm,
              bufs, stage_v, acc_sh, sems):
    cid = lax.axis_index("c")
    sid = lax.axis_index("s")
    tid = sid * _NC + cid
    ept = e // (_NC * _NS)
    nchunk = ept // _K
    nzw = 10
    rows_w = n // nzw
    ln = lax.iota(jnp.int32, 16)
    sem_g, sem_f, sem_s = sems

    @pl.when(sid < nzw)
    def _zero():
        def _zrow(r, _):
            stage_v[r, :] = jnp.zeros((16,), jnp.float32)
            return 0
        lax.fori_loop(0, rows_w, _zrow, 0)
        pltpu.sync_copy(stage_v, acc_sh.at[pl.ds(sid * rows_w, rows_w)])
    plsc.subcore_barrier()

    def _lin(j, b):
        base = tid * ept + j * _K
        ei_v, aep_v, _, _, _, _, _ = bufs[b]
        pltpu.sync_copy(ei_hbm.at[:, pl.ds(base, _K)], ei_v)
        pltpu.sync_copy(aep_hbm.at[pl.ds(base, _K)], aep_v)

    def _gath_start(b):
        ei_v, _, as_v, ad_v, _, _, _ = bufs[b]
        pltpu.async_copy(asp_hbm.at[ei_v.at[0]], as_v, sem_g[b])
        pltpu.async_copy(adp_hbm.at[ei_v.at[1]], ad_v, sem_g[b])

    def _gath_wait(b):
        ei_v, _, as_v, ad_v, _, _, _ = bufs[b]
        pltpu.make_async_copy(asp_hbm.at[ei_v.at[0]], as_v, sem_g[b]).wait()
        pltpu.make_async_copy(adp_hbm.at[ei_v.at[1]], ad_v, sem_g[b]).wait()

    def _outs_wait(b):
        _, _, _, _, scat_v, sflat_v, dsc_v = bufs[b]
        pltpu.make_async_copy(sflat_v, sflat_hbm.at[pl.ds(0, _K * 4)],
                              sem_f[b]).wait()
        pltpu.make_async_copy(scat_v, acc_sh.at[dsc_v], sem_s[b]).wait()

    def _compute(j, b, not_first):
        base = tid * ept + j * _K
        ei_v, aep_v, as_v, ad_v, scat_v, sflat_v, dsc_v = bufs[b]

        @pl.when(not_first)
        def _():
            _outs_wait(b)

        def _dcp(g, _):
            dsc_v[pl.ds(g * 16, 16)] = ei_v[1, pl.ds(g * 16, 16)]
            return 0
        lax.fori_loop(0, _K // 16, _dcp, 0)

        def _edge(k, _):
            aep = aep_v[k, :]
            t = as_v[k, :] + ad_v[k, :] + aep
            s16 = jnp.exp(_lrelu(t))
            ones16 = jnp.full((16,), 1.0, dtype=jnp.float32)
            zero16 = jnp.zeros((16,), jnp.float32)
            row = jnp.where(ln < 4, s16,
                            jnp.where(ln < 8, aep,
                                      jnp.where(ln == 8, ones16, zero16)))
            scat_v[k, :] = row
            return 0
        lax.fori_loop(0, _K, _edge, 0)

        def _pack(g, _):
            ridx = g * 4 + (ln >> 2)
            cidx = ln & 3
            sflat_v[pl.ds(g * 16, 16)] = plsc.load_gather(scat_v, [ridx, cidx])
            return 0
        lax.fori_loop(0, _K // 4, _pack, 0)

        pltpu.async_copy(sflat_v, sflat_hbm.at[pl.ds(base * 4, _K * 4)],
                         sem_f[b])
        pltpu.async_copy(scat_v, acc_sh.at[dsc_v], sem_s[b], add=True)

    _lin(0, 0)
    _gath_start(0)

    def _iter(i, _):
        for b in range(2):
            j = 2 * i + b

            @pl.when(j + 1 < nchunk)
            def _():
                _lin(j + 1, 1 - b)
                _gath_start(1 - b)

            @pl.when(j < nchunk)
            def _():
                _gath_wait(b)
                _compute(j, b, j >= 2)
        return 0
    lax.fori_loop(0, (nchunk + 1) // 2, _iter, 0)
    _outs_wait(0)
    _outs_wait(1)

    plsc.subcore_barrier()

    @pl.when(sid < nzw)
    def _read():
        r0 = sid * rows_w
        pltpu.sync_copy(acc_sh.at[pl.ds(r0, rows_w)], stage_v)
        pltpu.sync_copy(stage_v, part1_hbm.at[pl.ds(cid * n + r0, rows_w)])


def _sc2_body(n, e, ei_hbm, sflat_hbm, dinvp_hbm, xpcat_hbm,
              pout_hbm,
              bufs, zstage_v, out_sh, sems):
    cid = lax.axis_index("c")
    sid = lax.axis_index("s")
    tid = sid * _NC + cid
    ept = e // (_NC * _NS)
    nchunk = ept // _K
    nzw = 10
    rows_w = n // nzw
    zrows = zstage_v.shape[0]
    ln = lax.iota(jnp.int32, 16)
    sem_g, sem_s = sems

    def _lin(j, b):
        base = tid * ept + j * _K
        ei_v, _, _, sflat_v, _, _, _, _ = bufs[b]
        pltpu.sync_copy(ei_hbm.at[:, pl.ds(base, _K)], ei_v)
        pltpu.sync_copy(sflat_hbm.at[pl.ds(base * 4, _K * 4)], sflat_v)

    def _gath_start(p, b):
        ei_v, idx4_v, _, _, _, dinv_v, rows, _ = bufs[b]
        pltpu.async_copy(dinvp_hbm.at[ei_v.at[1]], dinv_v, sem_g[b])

        def _ix(g, _):
            v = ei_v[0, pl.ds(g * 16, 16)]
            for h in range(4):
                idx4_v[pl.ds(h * _K + g * 16, 16)] = v + (2 * h + p) * n
            return 0
        lax.fori_loop(0, _K // 16, _ix, 0)
        for h in range(4):
            pltpu.async_copy(xpcat_hbm.at[idx4_v.at[pl.ds(h * _K, _K)]],
                             rows[h], sem_g[b])

    def _gath_wait(p, b):
        ei_v, idx4_v, _, _, _, dinv_v, rows, _ = bufs[b]
        pltpu.make_async_copy(dinvp_hbm.at[ei_v.at[1]], dinv_v,
                              sem_g[b]).wait()
        for h in range(4):
            pltpu.make_async_copy(xpcat_hbm.at[idx4_v.at[pl.ds(h * _K, _K)]],
                                  rows[h], sem_g[b]).wait()

    def _scat_wait(b):
        _, _, dsc_v, _, _, _, _, msg_v = bufs[b]
        pltpu.make_async_copy(msg_v, out_sh.at[dsc_v], sem_s[b]).wait()

    def _compute(b, not_first):
        ei_v, _, dsc_v, sflat_v, w_v, dinv_v, rows, msg_v = bufs[b]

        def _wg(g, _):
            ridx = g * 4 + (ln >> 2)
            hidx = ln & 3
            dv = plsc.load_gather(dinv_v, [ridx, hidx])
            w_v[pl.ds(g * 16, 16)] = sflat_v[pl.ds(g * 16, 16)] * dv
            return 0
        lax.fori_loop(0, _K // 4, _wg, 0)

        @pl.when(not_first)
        def _():
            _scat_wait(b)

        def _dcp(g, _):
            dsc_v[pl.ds(g * 16, 16)] = ei_v[1, pl.ds(g * 16, 16)]
            return 0
        lax.fori_loop(0, _K // 16, _dcp, 0)

        def _edge(k, _):
            acc = [jnp.zeros((16,), jnp.float32) for _ in range(4)]
            for h in range(4):
                wvec = plsc.load_gather(
                    w_v, [jnp.full((16,), 4 * k + h, dtype=jnp.int32)])
                for g in range(4):
                    acc[g] = acc[g] + wvec * rows[h][k, pl.ds(g * 16, 16)]
            for g in range(4):
                msg_v[k, pl.ds(g * 16, 16)] = acc[g]
            return 0
        lax.fori_loop(0, _K, _edge, 0)
        pltpu.async_copy(msg_v, out_sh.at[dsc_v], sem_s[b], add=True)

    def _phase(p, _):
        @pl.when(sid < nzw)
        def _zero():
            def _zrow(r, _):
                for g in range(4):
                    zstage_v[r, pl.ds(g * 16, 16)] = jnp.zeros((16,),
                                                               jnp.float32)
                return 0
            lax.fori_loop(0, zrows, _zrow, 0)
            for j in range(rows_w // zrows):
                pltpu.sync_copy(
                    zstage_v,
                    out_sh.at[pl.ds(sid * rows_w + j * zrows, zrows)])
        plsc.subcore_barrier()

        _lin(0, 0)
        _gath_start(p, 0)

        def _iter(i, _):
            for b in range(2):
                j = 2 * i + b

                @pl.when(j + 1 < nchunk)
                def _():
                    _lin(j + 1, 1 - b)
                    _gath_start(p, 1 - b)

                @pl.when(j < nchunk)
                def _():
                    _gath_wait(p, b)
                    _compute(b, j >= 2)
            return 0
        lax.fori_loop(0, (nchunk + 1) // 2, _iter, 0)
        _scat_wait(0)
        _scat_wait(1)

        plsc.subcore_barrier()

        @pl.when(sid < nzw)
        def _read():
            rr = sid * rows_w
            for j in range(rows_w // zrows):
                pltpu.sync_copy(out_sh.at[pl.ds(rr + j * zrows, zrows)],
                                zstage_v)
                ro = (p * _NC + cid) * n + rr + j * zrows
                pltpu.sync_copy(zstage_v, pout_hbm.at[pl.ds(ro, zrows)])
        plsc.subcore_barrier()
        return 0
    lax.fori_loop(0, 2, _phase, 0)


def kernel(x, edge_index, edge_attr, W, att_src, att_dst, W_edge, att_edge,
           bias):
    n, d = x.shape
    e = edge_index.shape[1]
    h = att_src.shape[1]
    c = att_src.shape[2]
    ed = W_edge.shape[0]
    assert h == 4 and c == 128 and d == 128
    assert e % (_NC * _NS * _K) == 0 and n % 2000 == 0

    eye_h = jnp.eye(h, dtype=jnp.float32)
    m_src = (att_src[0][:, :, None] * eye_h[:, None, :]).reshape(h * c, h)
    m_dst = (att_dst[0][:, :, None] * eye_h[:, None, :]).reshape(h * c, h)
    m_edge = (att_edge[0][:, :, None] * eye_h[:, None, :]).reshape(h * c, h)

    bn = 1000
    gn = n // bn
    be = e // 40
    ge = e // be

    a1_out = pl.pallas_call(
        _a1_body,
        grid=(gn,),
        in_specs=[
            pl.BlockSpec((bn, d), lambda i: (i, 0)),
            pl.BlockSpec((d, h * c), lambda i: (0, 0)),
            pl.BlockSpec((h * c, h), lambda i: (0, 0)),
            pl.BlockSpec((h * c, h), lambda i: (0, 0)),
        ],
        out_specs=[pl.BlockSpec((bn, 64), lambda i: (i, 0))] * 8
        + [pl.BlockSpec((bn, 16), lambda i: (i, 0))] * 2,
        out_shape=[jax.ShapeDtypeStruct((n, 64), jnp.float32)] * 8
        + [jax.ShapeDtypeStruct((n, 16), jnp.float32)] * 2,
    )(x, W, m_src, m_dst)
    xps = a1_out[:8]
    asp, adp = a1_out[8], a1_out[9]

    aep = pl.pallas_call(
        _a2_body,
        grid=(ge,),
        in_specs=[
            pl.BlockSpec((be, ed), lambda i: (i, 0)),
            pl.BlockSpec((ed, h * c), lambda i: (0, 0)),
            pl.BlockSpec((h * c, h), lambda i: (0, 0)),
        ],
        out_specs=pl.BlockSpec((be, 16), lambda i: (i, 0)),
        out_shape=jax.ShapeDtypeStruct((e, 16), jnp.float32),
    )(edge_attr, W_edge, m_edge)

    mesh = plsc.VectorSubcoreMesh(core_axis_name="c", subcore_axis_name="s")
    sc_params = pltpu.CompilerParams(use_tc_tiling_on_sc=False,
                                     needs_layout_passes=False)

    sflat, part1 = pl.kernel(
        functools.partial(_sc1_body, n, e),
        out_type=(
            jax.ShapeDtypeStruct((4 * e,), jnp.float32),
            jax.ShapeDtypeStruct((_NC * n, 16), jnp.float32),
        ),
        mesh=mesh,
        compiler_params=sc_params,
        scratch_types=[
            tuple(
                (pltpu.VMEM((2, _K), jnp.int32),
                 pltpu.VMEM((_K, 16), jnp.float32),
                 pltpu.VMEM((_K, 16), jnp.float32),
                 pltpu.VMEM((_K, 16), jnp.float32),
                 pltpu.VMEM((_K, 16), jnp.float32),
                 pltpu.VMEM((4 * _K,), jnp.float32),
                 pltpu.VMEM((_K,), jnp.int32))
                for _ in range(2)),
            pltpu.VMEM((n // 10, 16), jnp.float32),
            pltpu.MemorySpace.VMEM_SHARED((n, 16), jnp.float32),
            ((pltpu.SemaphoreType.DMA, pltpu.SemaphoreType.DMA),
             (pltpu.SemaphoreType.DMA, pltpu.SemaphoreType.DMA),
             (pltpu.SemaphoreType.DMA, pltpu.SemaphoreType.DMA)),
        ],
    )(edge_index, aep, asp, adp)

    dinvp, oloop = pl.pallas_call(
        functools.partial(_b_body, h),
        grid=(gn,),
        in_specs=[
            pl.BlockSpec((bn, 16), lambda i: (i, 0)),
            pl.BlockSpec((bn, 16), lambda i: (i + gn, 0)),
            pl.BlockSpec((bn, 16), lambda i: (i, 0)),
            pl.BlockSpec((bn, 16), lambda i: (i, 0)),
        ] + [pl.BlockSpec((bn, 64), lambda i: (i, 0))] * 8,
        out_specs=[
            pl.BlockSpec((bn, 16), lambda i: (i, 0)),
            pl.BlockSpec((bn, c), lambda i: (i, 0)),
        ],
        out_shape=[
            jax.ShapeDtypeStruct((n, 16), jnp.float32),
            jax.ShapeDtypeStruct((n, c), jnp.float32),
        ],
    )(part1, part1, asp, adp, *xps)

    xpcat = jnp.concatenate(xps, axis=0)
    pout = pl.kernel(
        functools.partial(_sc2_body, n, e),
        out_type=jax.ShapeDtypeStruct((4 * n, 64), jnp.float32),
        mesh=mesh,
        compiler_params=sc_params,
        scratch_types=[
            tuple(
                (pltpu.VMEM((2, _K), jnp.int32),
                 pltpu.VMEM((4 * _K,), jnp.int32),
                 pltpu.VMEM((_K,), jnp.int32),
                 pltpu.VMEM((4 * _K,), jnp.float32),
                 pltpu.VMEM((4 * _K,), jnp.float32),
                 pltpu.VMEM((_K, 16), jnp.float32),
                 tuple(pltpu.VMEM((_K, 64), jnp.float32)
                       for _ in range(4)),
                 pltpu.VMEM((_K, 64), jnp.float32))
                for _ in range(2)),
            pltpu.VMEM((200, 64), jnp.float32),
            pltpu.MemorySpace.VMEM_SHARED((n, 64), jnp.float32),
            ((pltpu.SemaphoreType.DMA, pltpu.SemaphoreType.DMA),
             (pltpu.SemaphoreType.DMA, pltpu.SemaphoreType.DMA)),
        ],
    )(edge_index, sflat, dinvp, xpcat)

    out = pl.pallas_call(
        functools.partial(_c_body, 1.0 / h),
        grid=(gn,),
        in_specs=[
            pl.BlockSpec((bn, d), lambda i: (i, 0)),
            pl.BlockSpec((bn, 64), lambda i: (i, 0)),
            pl.BlockSpec((bn, 64), lambda i: (i + gn, 0)),
            pl.BlockSpec((bn, 64), lambda i: (i + 2 * gn, 0)),
            pl.BlockSpec((bn, 64), lambda i: (i + 3 * gn, 0)),
            pl.BlockSpec((bn, c), lambda i: (i, 0)),
            pl.BlockSpec((1, d), lambda i: (0, 0)),
        ],
        out_specs=pl.BlockSpec((bn, d), lambda i: (i, 0)),
        out_shape=jax.ShapeDtypeStruct((n, d), jnp.float32),
    )(x, pout, pout, pout, pout, oloop, bias.reshape(1, d))

    return out
